# baseline (speedup 1.0000x reference)
import functools
import jax
import jax.numpy as jnp
from jax.experimental import pallas as pl
from jax.experimental.pallas import tpu as pltpu

EPS = 1e-5


def _pass1_kernel(x_ref, w0_ref, w3_ref, ypre_ref, stats_ref, xsum_ref,
                  xs_ref, *, H, W, P, C, dils):
    rows = H * W
    xb = x_ref[0].astype(jnp.bfloat16)
    xc = jnp.transpose(xb)
    xs_ref[0:P] = jnp.zeros_like(xs_ref[0:P])
    xs_ref[P + H:] = jnp.zeros_like(xs_ref[P + H:])
    xs_ref[P:P + H, 0:P] = jnp.zeros_like(xs_ref[P:P + H, 0:P])
    xs_ref[P:P + H, P + W:] = jnp.zeros_like(xs_ref[P:P + H, P + W:])
    xs_ref[P:P + H, P:P + W, :] = xc.reshape(H, W, C)

    s0 = []
    q0 = []
    for h0 in range(0, H, 16):
        h1 = min(h0 + 16, H)
        y0 = jnp.dot(xc[h0 * W:h1 * W], w0_ref[...],
                     preferred_element_type=jnp.float32)
        ypre_ref[0, h0 * W:h1 * W, 0:C] = y0.astype(jnp.bfloat16)
        s0.append(jnp.sum(y0, axis=0, keepdims=True))
        q0.append(jnp.sum(y0 * y0, axis=0, keepdims=True))
    sums = [functools.reduce(lambda a, b: a + b, s0)]
    sqs = [functools.reduce(lambda a, b: a + b, q0)]

    for d, r in enumerate(dils):
        c0 = (1 + d) * C
        d_sums = []
        d_sqs = []
        if 2 * r < H:
            bands = ((0, r, 1, 3), (r, H - r, 0, 3), (H - r, H, 0, 2))
        else:
            bands = ((0, H, 0, 3),)
        MH = 8
        for b0, b1, k0, k1 in bands:
            for h0 in range(b0, b1, MH):
                h1 = min(h0 + MH, b1)
                taps = []
                for kh in range(k0, k1):
                    for kw in range(3):
                        oh = P + h0 + r * (kh - 1)
                        ow = P + r * (kw - 1)
                        taps.append(xs_ref[oh:oh + (h1 - h0), ow:ow + W, :])
                m = (h1 - h0) * W
                xt = jnp.concatenate(taps, axis=-1).reshape(m, (k1 - k0) * 3 * C)
                yd = jnp.dot(xt, w3_ref[d, k0 * 3 * C:k1 * 3 * C, :],
                             preferred_element_type=jnp.float32)
                ypre_ref[0, h0 * W:h1 * W, c0:c0 + C] = yd.astype(jnp.bfloat16)
                d_sums.append(jnp.sum(yd, axis=0, keepdims=True))
                d_sqs.append(jnp.sum(yd * yd, axis=0, keepdims=True))
        sums.append(functools.reduce(lambda a, b: a + b, d_sums))
        sqs.append(functools.reduce(lambda a, b: a + b, d_sqs))

    stats_ref[0, 0] = jnp.concatenate(
        [jnp.concatenate(sums, axis=-1), jnp.concatenate(sqs, axis=-1)], axis=0)
    xsum_ref[0, 0] = jnp.sum(xc.astype(jnp.float32), axis=0, keepdims=True)


def _pass2_kernel(ypre_ref, stats_ref, xsum_ref, wff_ref, wfi_ref, wi_ref,
                  gb_ref, gbi_ref, yt_ref, fstats_ref, wf_scr, const_scr,
                  *, N, HW):
    i = pl.program_id(0)

    @pl.when(i == 0)
    def _():
        cnt = float(N * HW)
        tot = jnp.sum(stats_ref[...], axis=0)
        mean_b = tot[0:1] / cnt
        var_b = tot[1:2] / cnt - mean_b * mean_b
        s_b = gb_ref[0:1] * jax.lax.rsqrt(var_b + EPS)
        shift_b = gb_ref[1:2] - mean_b * s_b
        wf_scr[...] = (wff_ref[...] * jnp.transpose(s_b)).astype(jnp.bfloat16)
        xmean = xsum_ref[...] / float(HW)
        yi = jnp.dot(xmean, wi_ref[...], preferred_element_type=jnp.float32)
        mi = jnp.sum(yi, axis=0, keepdims=True) / float(N)
        vi = jnp.sum((yi - mi) * (yi - mi), axis=0, keepdims=True) / float(N)
        yi_n = (yi - mi) * (gbi_ref[0:1] * jax.lax.rsqrt(vi + EPS)) + gbi_ref[1:2]
        cpart = jnp.sum(jnp.transpose(shift_b) * wff_ref[...],
                        axis=0, keepdims=True)
        const_scr[...] = cpart + jnp.dot(
            yi_n, wfi_ref[...], preferred_element_type=jnp.float32)

    y = jnp.dot(ypre_ref[0], wf_scr[...],
                preferred_element_type=jnp.float32) + const_scr[pl.ds(i, 1)]
    fstats_ref[0, 0] = jnp.concatenate(
        [jnp.sum(y, axis=0, keepdims=True),
         jnp.sum(y * y, axis=0, keepdims=True)], axis=0)
    yt_ref[0] = y.astype(jnp.bfloat16)


def kernel(x, w0, w3, b3, wi, wf, gamma, beta):
    del b3
    N, Cin, H, W = x.shape
    Cout = w0.shape[0]
    dils = (6, 12, 18)
    D = len(dils)
    P = max(dils)
    C1 = (D + 1) * Cout
    HW = H * W
    f32 = jnp.float32
    bf16 = jnp.bfloat16

    w0m = w0[:, :, 0, 0].T.astype(bf16)
    w3m = jnp.transpose(w3, (0, 3, 4, 2, 1)).reshape(D, 9 * Cin, Cout).astype(bf16)
    wim = wi[:, :, 0, 0].T.astype(f32)
    wf_per = wf[:, :, 0, 0].T.reshape(D + 2, Cout, Cout).astype(f32)

    cparams = pltpu.CompilerParams(
        dimension_semantics=("parallel",),
        vmem_limit_bytes=110 * 1024 * 1024)

    kernel1 = functools.partial(_pass1_kernel, H=H, W=W, P=P, C=Cin, dils=dils)
    ypre, stats, xsum = pl.pallas_call(
        kernel1,
        grid=(N,),
        in_specs=[
            pl.BlockSpec((1, Cin, HW), lambda n: (n, 0, 0)),
            pl.BlockSpec((Cin, Cout), lambda n: (0, 0)),
            pl.BlockSpec((D, 9 * Cin, Cout), lambda n: (0, 0, 0)),
        ],
        out_specs=(
            pl.BlockSpec((1, HW, C1), lambda n: (n, 0, 0)),
            pl.BlockSpec((1, 1, 2, C1), lambda n: (n, 0, 0, 0)),
            pl.BlockSpec((1, 1, 1, Cin), lambda n: (n, 0, 0, 0)),
        ),
        out_shape=(
            jax.ShapeDtypeStruct((N, HW, C1), bf16),
            jax.ShapeDtypeStruct((N, 1, 2, C1), f32),
            jax.ShapeDtypeStruct((N, 1, 1, Cin), f32),
        ),
        scratch_shapes=[pltpu.VMEM((H + 2 * P, W + 2 * P, Cin), bf16)],
        compiler_params=cparams,
    )(x.reshape(N, Cin, HW), w0m, w3m)

    cnt = float(N * HW)
    gb = jnp.stack([gamma[:D + 1].reshape(C1), beta[:D + 1].reshape(C1)])
    gbi = jnp.stack([gamma[D + 1], beta[D + 1]])
    kernel2 = functools.partial(_pass2_kernel, N=N, HW=HW)
    cparams2 = pltpu.CompilerParams(
        dimension_semantics=("arbitrary",),
        vmem_limit_bytes=110 * 1024 * 1024)
    yt, fstats = pl.pallas_call(
        kernel2,
        grid=(N,),
        in_specs=[
            pl.BlockSpec((1, HW, C1), lambda n: (n, 0, 0)),
            pl.BlockSpec((N, 2, C1), lambda n: (0, 0, 0)),
            pl.BlockSpec((N, Cin), lambda n: (0, 0)),
            pl.BlockSpec((C1, Cout), lambda n: (0, 0)),
            pl.BlockSpec((Cout, Cout), lambda n: (0, 0)),
            pl.BlockSpec((Cin, Cout), lambda n: (0, 0)),
            pl.BlockSpec((2, C1), lambda n: (0, 0)),
            pl.BlockSpec((2, Cout), lambda n: (0, 0)),
        ],
        out_specs=(
            pl.BlockSpec((1, HW, Cout), lambda n: (n, 0, 0)),
            pl.BlockSpec((1, 1, 2, Cout), lambda n: (n, 0, 0, 0)),
        ),
        out_shape=(
            jax.ShapeDtypeStruct((N, HW, Cout), bf16),
            jax.ShapeDtypeStruct((N, 1, 2, Cout), f32),
        ),
        scratch_shapes=[
            pltpu.VMEM((C1, Cout), bf16),
            pltpu.VMEM((N, Cout), f32),
        ],
        compiler_params=cparams2,
    )(ypre, stats.reshape(N, 2, C1), xsum.reshape(N, Cin),
      wf_per[:D + 1].reshape(C1, Cout), wf_per[D + 1], wim, gb, gbi)

    ftot = jnp.sum(fstats, axis=(0, 1))
    mf = ftot[0] / cnt
    vf = ftot[1] / cnt - mf * mf
    sf = gamma[D + 2] * jax.lax.rsqrt(vf + EPS)
    bf_ = beta[D + 2] - mf * sf
    out = yt.astype(f32) * sf + bf_
    return out.reshape(N, H, W, Cout).transpose(0, 3, 1, 2)


# speedup vs baseline: 1.0118x; 1.0118x over previous
import functools
import jax
import jax.numpy as jnp
from jax.experimental import pallas as pl
from jax.experimental.pallas import tpu as pltpu

EPS = 1e-5


def _pass1_kernel(x_ref, w0_ref, w3_ref, ypre_ref, stats_ref, xsum_ref,
                  xs_ref, *, H, W, P, C, dils):
    rows = H * W
    xb = x_ref[0].astype(jnp.bfloat16)
    xc = jnp.transpose(xb)
    xs_ref[0:P] = jnp.zeros_like(xs_ref[0:P])
    xs_ref[P + H:] = jnp.zeros_like(xs_ref[P + H:])
    xs_ref[P:P + H, 0:P] = jnp.zeros_like(xs_ref[P:P + H, 0:P])
    xs_ref[P:P + H, P + W:] = jnp.zeros_like(xs_ref[P:P + H, P + W:])
    xs_ref[P:P + H, P:P + W, :] = xc.reshape(H, W, C)

    s0 = []
    q0 = []
    for h0 in range(0, H, 16):
        h1 = min(h0 + 16, H)
        y0 = jnp.dot(xc[h0 * W:h1 * W], w0_ref[...],
                     preferred_element_type=jnp.float32)
        ypre_ref[0, h0 * W:h1 * W, 0:C] = y0.astype(jnp.bfloat16)
        s0.append(jnp.sum(y0, axis=0, keepdims=True))
        q0.append(jnp.sum(y0 * y0, axis=0, keepdims=True))
    sums = [functools.reduce(lambda a, b: a + b, s0)]
    sqs = [functools.reduce(lambda a, b: a + b, q0)]

    for d, r in enumerate(dils):
        c0 = (1 + d) * C
        d_sums = []
        d_sqs = []
        if 2 * r < H:
            bands = ((0, r, 1, 3), (r, H - r, 0, 3), (H - r, H, 0, 2))
        else:
            bands = ((0, H, 0, 3),)
        MH = 16
        for b0, b1, k0, k1 in bands:
            for h0 in range(b0, b1, MH):
                h1 = min(h0 + MH, b1)
                taps = []
                for kh in range(k0, k1):
                    for kw in range(3):
                        oh = P + h0 + r * (kh - 1)
                        ow = P + r * (kw - 1)
                        taps.append(xs_ref[oh:oh + (h1 - h0), ow:ow + W, :])
                m = (h1 - h0) * W
                xt = jnp.concatenate(taps, axis=-1).reshape(m, (k1 - k0) * 3 * C)
                yd = jnp.dot(xt, w3_ref[d, k0 * 3 * C:k1 * 3 * C, :],
                             preferred_element_type=jnp.float32)
                ypre_ref[0, h0 * W:h1 * W, c0:c0 + C] = yd.astype(jnp.bfloat16)
                d_sums.append(jnp.sum(yd, axis=0, keepdims=True))
                d_sqs.append(jnp.sum(yd * yd, axis=0, keepdims=True))
        sums.append(functools.reduce(lambda a, b: a + b, d_sums))
        sqs.append(functools.reduce(lambda a, b: a + b, d_sqs))

    stats_ref[0, 0] = jnp.concatenate(
        [jnp.concatenate(sums, axis=-1), jnp.concatenate(sqs, axis=-1)], axis=0)
    xsum_ref[0, 0] = jnp.sum(xc.astype(jnp.float32), axis=0, keepdims=True)


def _pass2_kernel(ypre_ref, stats_ref, xsum_ref, wff_ref, wfi_ref, wi_ref,
                  gb_ref, gbi_ref, yt_ref, fstats_ref, wf_scr, const_scr,
                  *, N, HW):
    i = pl.program_id(0)

    @pl.when(i == 0)
    def _():
        cnt = float(N * HW)
        tot = jnp.sum(stats_ref[...], axis=0)
        mean_b = tot[0:1] / cnt
        var_b = tot[1:2] / cnt - mean_b * mean_b
        s_b = gb_ref[0:1] * jax.lax.rsqrt(var_b + EPS)
        shift_b = gb_ref[1:2] - mean_b * s_b
        wf_scr[...] = (wff_ref[...] * jnp.transpose(s_b)).astype(jnp.bfloat16)
        xmean = xsum_ref[...] / float(HW)
        yi = jnp.dot(xmean, wi_ref[...], preferred_element_type=jnp.float32)
        mi = jnp.sum(yi, axis=0, keepdims=True) / float(N)
        vi = jnp.sum((yi - mi) * (yi - mi), axis=0, keepdims=True) / float(N)
        yi_n = (yi - mi) * (gbi_ref[0:1] * jax.lax.rsqrt(vi + EPS)) + gbi_ref[1:2]
        cpart = jnp.sum(jnp.transpose(shift_b) * wff_ref[...],
                        axis=0, keepdims=True)
        const_scr[...] = cpart + jnp.dot(
            yi_n, wfi_ref[...], preferred_element_type=jnp.float32)

    y = jnp.dot(ypre_ref[0], wf_scr[...],
                preferred_element_type=jnp.float32) + const_scr[pl.ds(i, 1)]
    fstats_ref[0, 0] = jnp.concatenate(
        [jnp.sum(y, axis=0, keepdims=True),
         jnp.sum(y * y, axis=0, keepdims=True)], axis=0)
    yt_ref[0] = y.astype(jnp.bfloat16)


def kernel(x, w0, w3, b3, wi, wf, gamma, beta):
    del b3
    N, Cin, H, W = x.shape
    Cout = w0.shape[0]
    dils = (6, 12, 18)
    D = len(dils)
    P = max(dils)
    C1 = (D + 1) * Cout
    HW = H * W
    f32 = jnp.float32
    bf16 = jnp.bfloat16

    w0m = w0[:, :, 0, 0].T.astype(bf16)
    w3m = jnp.transpose(w3, (0, 3, 4, 2, 1)).reshape(D, 9 * Cin, Cout).astype(bf16)
    wim = wi[:, :, 0, 0].T.astype(f32)
    wf_per = wf[:, :, 0, 0].T.reshape(D + 2, Cout, Cout).astype(f32)

    cparams = pltpu.CompilerParams(
        dimension_semantics=("parallel",),
        vmem_limit_bytes=110 * 1024 * 1024)

    kernel1 = functools.partial(_pass1_kernel, H=H, W=W, P=P, C=Cin, dils=dils)
    ypre, stats, xsum = pl.pallas_call(
        kernel1,
        grid=(N,),
        in_specs=[
            pl.BlockSpec((1, Cin, HW), lambda n: (n, 0, 0)),
            pl.BlockSpec((Cin, Cout), lambda n: (0, 0)),
            pl.BlockSpec((D, 9 * Cin, Cout), lambda n: (0, 0, 0)),
        ],
        out_specs=(
            pl.BlockSpec((1, HW, C1), lambda n: (n, 0, 0)),
            pl.BlockSpec((1, 1, 2, C1), lambda n: (n, 0, 0, 0)),
            pl.BlockSpec((1, 1, 1, Cin), lambda n: (n, 0, 0, 0)),
        ),
        out_shape=(
            jax.ShapeDtypeStruct((N, HW, C1), bf16),
            jax.ShapeDtypeStruct((N, 1, 2, C1), f32),
            jax.ShapeDtypeStruct((N, 1, 1, Cin), f32),
        ),
        scratch_shapes=[pltpu.VMEM((H + 2 * P, W + 2 * P, Cin), bf16)],
        compiler_params=cparams,
    )(x.reshape(N, Cin, HW), w0m, w3m)

    cnt = float(N * HW)
    gb = jnp.stack([gamma[:D + 1].reshape(C1), beta[:D + 1].reshape(C1)])
    gbi = jnp.stack([gamma[D + 1], beta[D + 1]])
    kernel2 = functools.partial(_pass2_kernel, N=N, HW=HW)
    cparams2 = pltpu.CompilerParams(
        dimension_semantics=("arbitrary",),
        vmem_limit_bytes=110 * 1024 * 1024)
    yt, fstats = pl.pallas_call(
        kernel2,
        grid=(N,),
        in_specs=[
            pl.BlockSpec((1, HW, C1), lambda n: (n, 0, 0)),
            pl.BlockSpec((N, 2, C1), lambda n: (0, 0, 0)),
            pl.BlockSpec((N, Cin), lambda n: (0, 0)),
            pl.BlockSpec((C1, Cout), lambda n: (0, 0)),
            pl.BlockSpec((Cout, Cout), lambda n: (0, 0)),
            pl.BlockSpec((Cin, Cout), lambda n: (0, 0)),
            pl.BlockSpec((2, C1), lambda n: (0, 0)),
            pl.BlockSpec((2, Cout), lambda n: (0, 0)),
        ],
        out_specs=(
            pl.BlockSpec((1, HW, Cout), lambda n: (n, 0, 0)),
            pl.BlockSpec((1, 1, 2, Cout), lambda n: (n, 0, 0, 0)),
        ),
        out_shape=(
            jax.ShapeDtypeStruct((N, HW, Cout), bf16),
            jax.ShapeDtypeStruct((N, 1, 2, Cout), f32),
        ),
        scratch_shapes=[
            pltpu.VMEM((C1, Cout), bf16),
            pltpu.VMEM((N, Cout), f32),
        ],
        compiler_params=cparams2,
    )(ypre, stats.reshape(N, 2, C1), xsum.reshape(N, Cin),
      wf_per[:D + 1].reshape(C1, Cout), wf_per[D + 1], wim, gb, gbi)

    ftot = jnp.sum(fstats, axis=(0, 1))
    mf = ftot[0] / cnt
    vf = ftot[1] / cnt - mf * mf
    sf = gamma[D + 2] * jax.lax.rsqrt(vf + EPS)
    bf_ = beta[D + 2] - mf * sf
    out = yt.astype(f32) * sf + bf_
    return out.reshape(N, H, W, Cout).transpose(0, 3, 1, 2)
